# baseline (speedup 1.0000x reference)
import functools

import jax
import jax.numpy as jnp
from jax import lax
from jax.experimental import pallas as pl
from jax.experimental.pallas import tpu as pltpu
from jax.experimental.pallas import tpu_sc as plsc

D_MODEL = 64
NUM_CORES = 2
NUM_SUBCORES = 16
NUM_WORKERS = NUM_CORES * NUM_SUBCORES
GATHER_W = 128
GROUP = 1024
J = GROUP // GATHER_W


@functools.cache
def _make_embed(B: int):
    assert B % (NUM_WORKERS * GROUP) == 0
    per_worker = B // NUM_WORKERS
    n_groups = per_worker // GROUP
    mesh = plsc.VectorSubcoreMesh(core_axis_name="c", subcore_axis_name="s")

    @functools.partial(
        pl.kernel,
        out_type=jax.ShapeDtypeStruct((B, D_MODEL), jnp.float32),
        mesh=mesh,
        scratch_types=[
            pltpu.VMEM((J, GATHER_W), jnp.int32),
            pltpu.VMEM((GROUP, D_MODEL), jnp.float32),
            pltpu.SemaphoreType.DMA,
        ],
        compiler_params=pltpu.CompilerParams(use_tc_tiling_on_sc=False),
    )
    def embed(idx_hbm, table_hbm, out_hbm, idx_v, rows_v, gsem):
        wid = lax.axis_index("s") * NUM_CORES + lax.axis_index("c")
        base = wid * per_worker

        @pl.loop(0, n_groups)
        def group(g):
            off = base + g * GROUP
            row = pl.multiple_of(off // GATHER_W, 8)
            pltpu.sync_copy(idx_hbm.at[pl.ds(row, J), :], idx_v)
            for j in range(J):
                pltpu.async_copy(
                    table_hbm.at[idx_v.at[j]],
                    rows_v.at[pl.ds(j * GATHER_W, GATHER_W), :],
                    gsem,
                )
            for j in range(J):
                pltpu.make_async_copy(
                    table_hbm.at[idx_v.at[j]],
                    rows_v.at[pl.ds(j * GATHER_W, GATHER_W), :],
                    gsem,
                ).wait()
            pltpu.sync_copy(rows_v, out_hbm.at[pl.ds(off, GROUP), :])

    return embed


def kernel(x, table):
    B = x.size
    idx2 = x.reshape(B // GATHER_W, GATHER_W)
    out = _make_embed(B)(idx2, table)
    return out.reshape(x.shape + (D_MODEL,))


# speedup vs baseline: 1.0304x; 1.0304x over previous
import functools

import jax
import jax.numpy as jnp
from jax import lax
from jax.experimental import pallas as pl
from jax.experimental.pallas import tpu as pltpu
from jax.experimental.pallas import tpu_sc as plsc

D_MODEL = 64
NUM_CORES = 2
NUM_SUBCORES = 16
NUM_WORKERS = NUM_CORES * NUM_SUBCORES
GATHER_W = 128
GROUP = 512
J = GROUP // GATHER_W


@functools.cache
def _make_embed(B: int):
    assert B % (NUM_WORKERS * 2 * GROUP) == 0
    per_worker = B // NUM_WORKERS
    n_groups = per_worker // GROUP
    mesh = plsc.VectorSubcoreMesh(core_axis_name="c", subcore_axis_name="s")

    @functools.partial(
        pl.kernel,
        out_type=jax.ShapeDtypeStruct((B, D_MODEL), jnp.float32),
        mesh=mesh,
        scratch_types=[
            pltpu.VMEM((2, J, GATHER_W), jnp.int32),
            pltpu.VMEM((2, GROUP, D_MODEL), jnp.float32),
            [pltpu.SemaphoreType.DMA] * 2,
            [pltpu.SemaphoreType.DMA] * 2,
            [pltpu.SemaphoreType.DMA] * 2,
        ],
        compiler_params=pltpu.CompilerParams(use_tc_tiling_on_sc=False),
    )
    def embed(idx_hbm, table_hbm, out_hbm, idx_v, rows_v, isem, gsem, ssem):
        wid = lax.axis_index("s") * NUM_CORES + lax.axis_index("c")
        base = wid * per_worker

        def idx_copy(g, b):
            row = pl.multiple_of((base + g * GROUP) // GATHER_W, GROUP // GATHER_W)
            return pltpu.make_async_copy(
                idx_hbm.at[pl.ds(row, J), :], idx_v.at[b], isem[b])

        def gather_copy(b, j):
            return pltpu.make_async_copy(
                table_hbm.at[idx_v.at[b, j]],
                rows_v.at[b, pl.ds(j * GATHER_W, GATHER_W), :],
                gsem[b],
            )

        def store_copy(g, b):
            return pltpu.make_async_copy(
                rows_v.at[b], out_hbm.at[pl.ds(base + g * GROUP, GROUP), :],
                ssem[b],
            )

        idx_copy(0, 0).start()
        idx_copy(0, 0).wait()

        @pl.loop(0, n_groups, step=2)
        def sup(t):
            for b in (0, 1):
                g = t + b

                @pl.when(g >= 1)
                def _():
                    idx_copy(g, b).wait()

                @pl.when(g >= 2)
                def _():
                    store_copy(g - 2, b).wait()

                for j in range(J):
                    gather_copy(b, j).start()

                @pl.when(g >= 1)
                def _():
                    for j in range(J):
                        gather_copy(1 - b, j).wait()
                    store_copy(g - 1, 1 - b).start()

                @pl.when(g + 1 < n_groups)
                def _():
                    idx_copy(g + 1, 1 - b).start()

        for j in range(J):
            gather_copy(1, j).wait()
        store_copy(n_groups - 1, 1).start()
        store_copy(n_groups - 2, 0).wait()
        store_copy(n_groups - 1, 1).wait()

    return embed


def kernel(x, table):
    B = x.size
    idx2 = x.reshape(B // GATHER_W, GATHER_W)
    out = _make_embed(B)(idx2, table)
    return out.reshape(x.shape + (D_MODEL,))
